# baseline (speedup 1.0000x reference)
import functools
import jax
import jax.numpy as jnp
from jax import lax
from jax.experimental import pallas as pl
from jax.experimental.pallas import tpu as pltpu
from jax.experimental.pallas import tpu_sc as plsc

BATCH = 16384
HIST = 50
DIM = 32
BLK = 128
DHI = DIM // 8

_info = plsc.get_sparse_core_info()
NC, NS = _info.num_cores, _info.num_subcores
NW = NC * NS
NBLK = BATCH // BLK
BLK_PER_W = NBLK // NW

_mesh = plsc.VectorSubcoreMesh(core_axis_name="c", subcore_axis_name="s")


@functools.partial(
    pl.kernel,
    mesh=_mesh,
    out_type=jax.ShapeDtypeStruct((HIST, DHI, NBLK, 8, BLK), jnp.float32),
    scratch_types=[
        pltpu.VMEM((HIST, BLK_PER_W, BLK), jnp.int32),
        pltpu.VMEM((2, BLK, DIM), jnp.float32),
        pltpu.VMEM((2, DHI, 8, BLK), jnp.float32),
        pltpu.SemaphoreType.DMA,
        pltpu.SemaphoreType.DMA,
        pltpu.SemaphoreType.DMA,
        pltpu.SemaphoreType.DMA,
    ],
    compiler_params=pltpu.CompilerParams(use_tc_tiling_on_sc=False,
                                         needs_layout_passes=False),
)
def _gather_kernel(idx_hbm, table_hbm, out_hbm, idx_v, rv, rt,
                   sem_g0, sem_g1, sem_o0, sem_o1):
    wid = lax.axis_index("s") * NC + lax.axis_index("c")
    iota = lax.iota(jnp.int32, 16)
    rows16 = [iota + 16 * g for g in range(BLK // 16)]

    pltpu.sync_copy(idx_hbm.at[:, pl.ds(wid * BLK_PER_W, BLK_PER_W)], idx_v)

    def fire(h, blk, p, sem):
        pltpu.async_copy(table_hbm.at[idx_v.at[h, blk]], rv.at[p], sem)

    def drain_gather(p, sem):
        pltpu.make_async_copy(table_hbm.at[idx_v.at[0, 0]], rv.at[p],
                              sem).wait()

    def transpose(p):
        for d in range(DIM):
            col = jnp.full((16,), d, jnp.int32)
            for g in range(BLK // 16):
                v = plsc.load_gather(rv.at[p], [rows16[g], col])
                rt[p, d // 8, d % 8, pl.ds(16 * g, 16)] = v

    def fire_out(h, blk, p, sem):
        pltpu.async_copy(rt.at[p], out_hbm.at[h, :, wid * BLK_PER_W + blk],
                         sem)

    def drain_out(h, blk, p, sem):
        pltpu.make_async_copy(rt.at[p],
                              out_hbm.at[0, :, wid * BLK_PER_W], sem).wait()

    def blk_body(blk, carry):
        fire(0, blk, 0, sem_g0)

        def body(i, carry2):
            h = 2 * i
            fire(h + 1, blk, 1, sem_g1)
            drain_gather(0, sem_g0)
            transpose(0)
            fire_out(h, blk, 0, sem_o0)

            @pl.when(i < HIST // 2 - 1)
            def _():
                fire(h + 2, blk, 0, sem_g0)

            drain_gather(1, sem_g1)
            transpose(1)
            fire_out(h + 1, blk, 1, sem_o1)
            drain_out(h, blk, 0, sem_o0)
            drain_out(h + 1, blk, 1, sem_o1)
            return carry2

        lax.fori_loop(0, HIST // 2, body, 0)
        return carry

    lax.fori_loop(0, BLK_PER_W, blk_body, 0)


def kernel(token_ids, weight):
    tids = token_ids.astype(jnp.int32).T.reshape(HIST, NBLK, BLK)
    out5d = _gather_kernel(tids, weight)
    return out5d.transpose(2, 4, 0, 1, 3).reshape(BATCH, HIST, DIM)


# speedup vs baseline: 1.3900x; 1.3900x over previous
import functools
import jax
import jax.numpy as jnp
from jax import lax
from jax.experimental import pallas as pl
from jax.experimental.pallas import tpu as pltpu
from jax.experimental.pallas import tpu_sc as plsc

BATCH = 16384
HIST = 50
DIM = 32
BLK = 128
DHI = DIM // 8

_info = plsc.get_sparse_core_info()
NC, NS = _info.num_cores, _info.num_subcores
NW = NC * NS
NBLK = BATCH // BLK
BLK_PER_W = NBLK // NW

_mesh = plsc.VectorSubcoreMesh(core_axis_name="c", subcore_axis_name="s")


@functools.partial(
    pl.kernel,
    mesh=_mesh,
    out_type=jax.ShapeDtypeStruct((HIST, DHI, NBLK, 8, BLK), jnp.float32),
    scratch_types=[
        pltpu.VMEM((HIST, BLK_PER_W, BLK), jnp.int32),
        pltpu.VMEM((2, BLK, DIM), jnp.float32),
        pltpu.VMEM((2, DHI, 8, BLK), jnp.float32),
        pltpu.SemaphoreType.DMA,
        pltpu.SemaphoreType.DMA,
        pltpu.SemaphoreType.DMA,
        pltpu.SemaphoreType.DMA,
    ],
    compiler_params=pltpu.CompilerParams(use_tc_tiling_on_sc=False,
                                         needs_layout_passes=False),
)
def _gather_kernel(idx_hbm, table_hbm, out_hbm, idx_v, rv, rt,
                   sem_g0, sem_g1, sem_o0, sem_o1):
    wid = lax.axis_index("s") * NC + lax.axis_index("c")
    iota = lax.iota(jnp.int32, 16)
    rows16 = [iota + 16 * g for g in range(BLK // 16)]
    colmod = [(iota + k) & 15 for k in range(16)]

    pltpu.sync_copy(idx_hbm.at[:, pl.ds(wid * BLK_PER_W, BLK_PER_W)], idx_v)

    def fire(h, blk, p, sem):
        pltpu.async_copy(table_hbm.at[idx_v.at[h, blk]], rv.at[p], sem)

    def drain_gather(p, sem):
        pltpu.make_async_copy(table_hbm.at[idx_v.at[0, 0]], rv.at[p],
                              sem).wait()

    def transpose(p):
        for d0 in (0, 16):
            for k in range(16):
                dvec = colmod[k] | d0
                dhi = dvec >> 3
                dlo = dvec & 7
                for g in range(BLK // 16):
                    v = plsc.load_gather(rv.at[p], [rows16[g], dvec])
                    plsc.store_scatter(rt.at[p], [dhi, dlo, rows16[g]], v)

    def fire_out(h, blk, p, sem):
        pltpu.async_copy(rt.at[p], out_hbm.at[h, :, wid * BLK_PER_W + blk],
                         sem)

    def drain_out(h, blk, p, sem):
        pltpu.make_async_copy(rt.at[p],
                              out_hbm.at[0, :, wid * BLK_PER_W], sem).wait()

    def blk_body(blk, carry):
        fire(0, blk, 0, sem_g0)

        def body(i, carry2):
            h = 2 * i
            fire(h + 1, blk, 1, sem_g1)
            drain_gather(0, sem_g0)
            transpose(0)
            fire_out(h, blk, 0, sem_o0)

            @pl.when(i < HIST // 2 - 1)
            def _():
                fire(h + 2, blk, 0, sem_g0)

            drain_gather(1, sem_g1)
            transpose(1)
            fire_out(h + 1, blk, 1, sem_o1)
            drain_out(h, blk, 0, sem_o0)
            drain_out(h + 1, blk, 1, sem_o1)
            return carry2

        lax.fori_loop(0, HIST // 2, body, 0)
        return carry

    lax.fori_loop(0, BLK_PER_W, blk_body, 0)


def kernel(token_ids, weight):
    tids = token_ids.astype(jnp.int32).T.reshape(HIST, NBLK, BLK)
    out5d = _gather_kernel(tids, weight)
    return out5d.transpose(2, 4, 0, 1, 3).reshape(BATCH, HIST, DIM)


# speedup vs baseline: 1.6642x; 1.1973x over previous
import functools
import jax
import jax.numpy as jnp
from jax import lax
from jax.experimental import pallas as pl
from jax.experimental.pallas import tpu as pltpu
from jax.experimental.pallas import tpu_sc as plsc

BATCH = 16384
HIST = 50
DIM = 32
BLK = 128
DHI = DIM // 8

_info = plsc.get_sparse_core_info()
NC, NS = _info.num_cores, _info.num_subcores
NW = NC * NS
NBLK = BATCH // BLK
BLK_PER_W = NBLK // NW

_mesh = plsc.VectorSubcoreMesh(core_axis_name="c", subcore_axis_name="s")


@functools.partial(
    pl.kernel,
    mesh=_mesh,
    out_type=jax.ShapeDtypeStruct((HIST, DHI, NBLK, 8, BLK), jnp.float32),
    scratch_types=[
        pltpu.VMEM((HIST, BLK_PER_W, BLK), jnp.int32),
        pltpu.VMEM((2, BLK, DIM), jnp.float32),
        pltpu.VMEM((2, DHI, 8, BLK), jnp.float32),
        pltpu.SemaphoreType.DMA,
        pltpu.SemaphoreType.DMA,
        pltpu.SemaphoreType.DMA,
        pltpu.SemaphoreType.DMA,
    ],
    compiler_params=pltpu.CompilerParams(use_tc_tiling_on_sc=False,
                                         needs_layout_passes=False),
)
def _gather_kernel(idx_hbm, table_hbm, out_hbm, idx_v, rv, rt,
                   sem_g0, sem_g1, sem_o0, sem_o1):
    wid = lax.axis_index("s") * NC + lax.axis_index("c")
    iota = lax.iota(jnp.int32, 16)
    rows16 = [iota + 16 * g for g in range(BLK // 16)]

    pltpu.sync_copy(idx_hbm.at[:, pl.ds(wid * BLK_PER_W, BLK_PER_W)], idx_v)

    def fire(h, blk, p, sem):
        pltpu.async_copy(table_hbm.at[idx_v.at[h, blk]], rv.at[p], sem)

    def drain_gather(p, sem):
        pltpu.make_async_copy(table_hbm.at[idx_v.at[0, 0]], rv.at[p],
                              sem).wait()

    def transpose(p):
        def jbody(j, carry3):
            dvec = ((iota + (j & 15)) & 15) | ((j >> 4) << 4)
            dhi = dvec >> 3
            dlo = dvec & 7
            for g in range(BLK // 16):
                v = plsc.load_gather(rv.at[p], [rows16[g], dvec])
                plsc.store_scatter(rt.at[p], [dhi, dlo, rows16[g]], v)
            return carry3

        lax.fori_loop(0, 2 * 16, jbody, 0)

    def fire_out(h, blk, p, sem):
        pltpu.async_copy(rt.at[p], out_hbm.at[h, :, wid * BLK_PER_W + blk],
                         sem)

    def drain_out(h, blk, p, sem):
        pltpu.make_async_copy(rt.at[p],
                              out_hbm.at[0, :, wid * BLK_PER_W], sem).wait()

    def blk_body(blk, carry):
        fire(0, blk, 0, sem_g0)

        def body(i, carry2):
            h = 2 * i
            fire(h + 1, blk, 1, sem_g1)
            drain_gather(0, sem_g0)
            transpose(0)
            fire_out(h, blk, 0, sem_o0)

            @pl.when(i < HIST // 2 - 1)
            def _():
                fire(h + 2, blk, 0, sem_g0)

            drain_gather(1, sem_g1)
            transpose(1)
            fire_out(h + 1, blk, 1, sem_o1)
            drain_out(h, blk, 0, sem_o0)
            drain_out(h + 1, blk, 1, sem_o1)
            return carry2

        lax.fori_loop(0, HIST // 2, body, 0)
        return carry

    lax.fori_loop(0, BLK_PER_W, blk_body, 0)


def kernel(token_ids, weight):
    tids = token_ids.astype(jnp.int32).T.reshape(HIST, NBLK, BLK)
    out5d = _gather_kernel(tids, weight)
    return out5d.transpose(2, 4, 0, 1, 3).reshape(BATCH, HIST, DIM)


# speedup vs baseline: 2.3733x; 1.4261x over previous
import functools
import jax
import jax.numpy as jnp
from jax import lax
from jax.experimental import pallas as pl
from jax.experimental.pallas import tpu as pltpu
from jax.experimental.pallas import tpu_sc as plsc

BATCH = 16384
HIST = 50
DIM = 32
TOKENS = 1000000
BLK = 128
DHI = DIM // 8

_info = plsc.get_sparse_core_info()
NC, NS = _info.num_cores, _info.num_subcores
NW = NC * NS
NBLK = BATCH // BLK
BLK_PER_W = NBLK // NW

NCH = TOKENS // BLK
REM = TOKENS - NCH * BLK
OROWS = TOKENS * DIM // BLK

_mesh = plsc.VectorSubcoreMesh(core_axis_name="c", subcore_axis_name="s")


@functools.partial(
    pl.kernel,
    mesh=_mesh,
    out_type=jax.ShapeDtypeStruct((OROWS, BLK), jnp.float32),
    scratch_types=[
        pltpu.VMEM((2, DIM, BLK), jnp.float32),
        pltpu.VMEM((2, DIM, BLK), jnp.float32),
        pltpu.SemaphoreType.DMA,
        pltpu.SemaphoreType.DMA,
        pltpu.SemaphoreType.DMA,
        pltpu.SemaphoreType.DMA,
    ],
    compiler_params=pltpu.CompilerParams(use_tc_tiling_on_sc=True,
                                         needs_layout_passes=False),
)
def _convert_kernel(wt_hbm, tail_hbm, out_hbm, sv, dv,
                    sem_i0, sem_i1, sem_o0, sem_o1):
    wid = lax.axis_index("s") * NC + lax.axis_index("c")
    iota = lax.iota(jnp.int32, 16)
    rows_g = [(iota + 16 * g) >> 2 for g in range(BLK // 16)]
    kkbase = (iota & 3) * 32
    sem_i = (sem_i0, sem_i1)
    sem_o = (sem_o0, sem_o1)
    extra = wid < NCH - NW * (NCH // NW)

    def fire_in(k, p):
        c = wid + NW * k
        pltpu.async_copy(wt_hbm.at[:, pl.ds(c * BLK, BLK)], sv.at[p],
                         sem_i[p])

    def drain_in(p):
        pltpu.make_async_copy(wt_hbm.at[:, pl.ds(0, BLK)], sv.at[p],
                              sem_i[p]).wait()

    def transpose(p, ng):
        def jbody(j, carry):
            dvec = ((iota + (j & 15)) & 15) | ((j >> 4) << 4)
            kvec = kkbase + dvec
            for g in range(ng):
                v = plsc.load_gather(sv.at[p], [dvec, iota + 16 * g])
                plsc.store_scatter(dv.at[p], [rows_g[g], kvec], v)
            return carry

        lax.fori_loop(0, DIM, jbody, 0)

    def fire_out(k, p):
        c = wid + NW * k
        pltpu.async_copy(dv.at[p], out_hbm.at[pl.ds(c * DIM, DIM)], sem_o[p])

    def drain_out(p):
        pltpu.make_async_copy(dv.at[p], out_hbm.at[pl.ds(0, DIM)],
                              sem_o[p]).wait()

    KMIN = NCH // NW
    fire_in(0, 0)

    def body(t, carry):
        k = 2 * t
        fire_in(k + 1, 1)
        drain_in(0)

        @pl.when(t >= 1)
        def _():
            drain_out(0)

        transpose(0, BLK // 16)
        fire_out(k, 0)

        @pl.when(jnp.logical_or(t < KMIN // 2 - 1, extra))
        def _():
            fire_in(k + 2, 0)

        drain_in(1)

        @pl.when(t >= 1)
        def _():
            drain_out(1)

        transpose(1, BLK // 16)
        fire_out(k + 1, 1)
        return carry

    lax.fori_loop(0, KMIN // 2, body, 0)
    drain_out(0)
    drain_out(1)

    @pl.when(extra)
    def _():
        drain_in(0)
        transpose(0, BLK // 16)
        fire_out(KMIN, 0)
        drain_out(0)

    @pl.when(wid == NW - 1)
    def _():
        pltpu.sync_copy(tail_hbm, sv.at[0])
        transpose(0, REM // 16)
        pltpu.sync_copy(dv.at[0, pl.ds(0, REM * DIM // BLK)],
                        out_hbm.at[pl.ds(NCH * DIM, REM * DIM // BLK)])


@functools.partial(
    pl.kernel,
    mesh=_mesh,
    out_type=jax.ShapeDtypeStruct((HIST, DHI, NBLK, 8, BLK), jnp.float32),
    scratch_types=[
        pltpu.VMEM((HIST, BLK_PER_W, BLK), jnp.int32),
        pltpu.VMEM((2, BLK, DIM), jnp.float32),
        pltpu.VMEM((2, DHI, 8, BLK), jnp.float32),
        pltpu.SemaphoreType.DMA,
        pltpu.SemaphoreType.DMA,
        pltpu.SemaphoreType.DMA,
        pltpu.SemaphoreType.DMA,
    ],
    compiler_params=pltpu.CompilerParams(use_tc_tiling_on_sc=False,
                                         needs_layout_passes=False),
)
def _gather_kernel(idx_hbm, table_hbm, out_hbm, idx_v, rv, rt,
                   sem_g0, sem_g1, sem_o0, sem_o1):
    wid = lax.axis_index("s") * NC + lax.axis_index("c")
    iota = lax.iota(jnp.int32, 16)
    rows16 = [iota + 16 * g for g in range(BLK // 16)]

    pltpu.sync_copy(idx_hbm.at[:, pl.ds(wid * BLK_PER_W, BLK_PER_W)], idx_v)

    def fire(h, blk, p, sem):
        pltpu.async_copy(table_hbm.at[idx_v.at[h, blk]], rv.at[p], sem)

    def drain_gather(p, sem):
        pltpu.make_async_copy(table_hbm.at[idx_v.at[0, 0]], rv.at[p],
                              sem).wait()

    def transpose(p):
        def jbody(j, carry3):
            dvec = ((iota + (j & 15)) & 15) | ((j >> 4) << 4)
            dhi = dvec >> 3
            dlo = dvec & 7
            for g in range(BLK // 16):
                v = plsc.load_gather(rv.at[p], [rows16[g], dvec])
                plsc.store_scatter(rt.at[p], [dhi, dlo, rows16[g]], v)
            return carry3

        lax.fori_loop(0, 2 * 16, jbody, 0)

    def fire_out(h, blk, p, sem):
        pltpu.async_copy(rt.at[p], out_hbm.at[h, :, wid * BLK_PER_W + blk],
                         sem)

    def drain_out(h, blk, p, sem):
        pltpu.make_async_copy(rt.at[p],
                              out_hbm.at[0, :, wid * BLK_PER_W], sem).wait()

    def blk_body(blk, carry):
        fire(0, blk, 0, sem_g0)

        def body(i, carry2):
            h = 2 * i
            fire(h + 1, blk, 1, sem_g1)
            drain_gather(0, sem_g0)
            transpose(0)
            fire_out(h, blk, 0, sem_o0)

            @pl.when(i < HIST // 2 - 1)
            def _():
                fire(h + 2, blk, 0, sem_g0)

            drain_gather(1, sem_g1)
            transpose(1)
            fire_out(h + 1, blk, 1, sem_o1)
            drain_out(h, blk, 0, sem_o0)
            drain_out(h + 1, blk, 1, sem_o1)
            return carry2

        lax.fori_loop(0, HIST // 2, body, 0)
        return carry

    lax.fori_loop(0, BLK_PER_W, blk_body, 0)


def kernel(token_ids, weight):
    tids = token_ids.astype(jnp.int32).T.reshape(HIST, NBLK, BLK)
    tail = jnp.zeros((DIM, BLK), jnp.float32).at[:, :REM].set(
        weight[NCH * BLK:].T)
    table = _convert_kernel(weight.T, tail).reshape(TOKENS, DIM)
    out5d = _gather_kernel(tids, table)
    return out5d.transpose(2, 4, 0, 1, 3).reshape(BATCH, HIST, DIM)
